# baseline (speedup 1.0000x reference)
import functools

import jax
import jax.numpy as jnp
from jax import lax
from jax.experimental import pallas as pl
from jax.experimental.pallas import tpu as pltpu
from jax.experimental.pallas import tpu_sc as plsc

N = 10000
E = 320000
D = 128
ED = 16
G = 64

NC = 2
NS = 16
NW = NC * NS
C = 64
E_PAD = 327680
EPW0 = 12288
EPW1 = 8192
CH0 = EPW0 // C
CH1 = EPW1 // C
NPAD = 10240
ROWS_PER_SUB = NPAD // NS
DUMMY = N

NB = 1000
NGRID = N // NB


def _elu(u):
    return jnp.where(u > 0, u, jnp.exp(jnp.minimum(u, 0.0)) - 1.0)


EB = 1024


def _proj_body(ea_ref, w_ref, b_ref, *outs):
    u = jnp.dot(ea_ref[...], w_ref[...], preferred_element_type=jnp.float32)
    u = u + b_ref[...]
    for q, o_ref in enumerate(outs):
        o_ref[...] = u[:, q * 128:(q + 1) * 128]


def _project(ea_pad, wt, b, width):
    grid = E_PAD // EB
    nout = width // 128
    out = jax.ShapeDtypeStruct((E_PAD, 128), jnp.float32)
    res = pl.pallas_call(
        _proj_body,
        grid=(grid,),
        in_specs=[
            pl.BlockSpec((EB, ED), lambda i: (i, 0)),
            pl.BlockSpec((ED, width), lambda i: (0, 0)),
            pl.BlockSpec((1, width), lambda i: (0, 0)),
        ],
        out_specs=[pl.BlockSpec((EB, 128), lambda i: (i, 0))] * nout,
        out_shape=[out] * nout,
    )(ea_pad, wt, b)
    return res


def _edge_body(h_hbm, ea_hbm, src_hbm, dst_hbm, out_hbm, spm,
               si0, si1, si2, si3, di0, di1, di2, di3,
               rows0, rows1, eab0, eab1,
               is0, is1, is2, is3, ds0, ds1, ds2, ds3,
               gsem0, gsem1, esem0, esem1, ssem0, ssem1):
    cid = lax.axis_index("c")
    sid = lax.axis_index("s")
    wbase = jnp.where(cid == 0, sid * EPW0, NS * EPW0 + sid * EPW1)
    nch = jnp.where(cid == 0, CH0, CH1)
    sidx = (si0, si1, si2, si3)
    didx = (di0, di1, di2, di3)
    isem = (is0, is1, is2, is3)
    dsem = (ds0, ds1, ds2, ds3)
    rows = (rows0, rows1)
    eab = (eab0, eab1)
    gsem = (gsem0, gsem1)
    esem = (esem0, esem1)
    ssem = (ssem0, ssem1)

    def _idxload(j, s):
        base = wbase + j * C
        a = pltpu.make_async_copy(src_hbm.at[pl.ds(base, C)], sidx[s], isem[s])
        b = pltpu.make_async_copy(dst_hbm.at[pl.ds(base, C)], didx[s], dsem[s])
        return a, b

    def _gather_s(s, b):
        return pltpu.make_async_copy(h_hbm.at[sidx[s]], rows[b], gsem[b])

    def _eacopy(j, b):
        return pltpu.make_async_copy(
            ea_hbm.at[pl.ds(wbase + j * C, C), :], eab[b], esem[b])

    def _scatter(s, b):
        return pltpu.make_async_copy(rows[b], spm.at[didx[s]], ssem[b])

    a0, b0 = _idxload(0, 0)
    a0.start()
    b0.start()
    a1, b1 = _idxload(1, 1)
    a1.start()
    b1.start()

    zeros16 = jnp.zeros((16,), jnp.float32)

    def zrow(i, carry):
        for k in range(8):
            rows0[i, pl.ds(k * 16, 16)] = zeros16
        return carry

    lax.fori_loop(0, C, zrow, 0)

    r0 = sid * ROWS_PER_SUB

    def zspm(r, carry):
        pltpu.sync_copy(rows0, spm.at[pl.ds(r0 + r * C, C), :])
        return carry

    lax.fori_loop(0, ROWS_PER_SUB // C, zspm, 0)
    plsc.subcore_barrier()

    a0.wait()
    _gather_s(0, 0).start()
    _eacopy(0, 0).start()

    def quad(jq, carry):
        for s in range(4):
            j = jq * 4 + s
            b = s % 2
            nb = 1 - b
            ns = (s + 1) % 4

            @pl.when(j >= 1)
            def _():
                _scatter((s + 3) % 4, nb).wait()

            @pl.when(j < nch - 2)
            def _():
                a, bb = _idxload(j + 2, (s + 2) % 4)
                a.start()
                bb.start()

            @pl.when(j < nch - 1)
            def _():
                pltpu.make_async_copy(
                    src_hbm.at[pl.ds(wbase + (j + 1) * C, C)],
                    sidx[ns], isem[ns]).wait()
                _gather_s(ns, nb).start()
                _eacopy(j + 1, nb).start()

            _gather_s(s, b).wait()
            _eacopy(j, b).wait()

            rb = rows[b]
            eb = eab[b]

            def comp(i, inner):
                for k in range(8):
                    sl = pl.ds(k * 16, 16)
                    rb[i, sl] = jnp.maximum(rb[i, sl] + eb[i, sl], 0.0)
                return inner

            lax.fori_loop(0, C, comp, 0)

            pltpu.make_async_copy(
                dst_hbm.at[pl.ds(wbase + j * C, C)], didx[s], dsem[s]).wait()
            pltpu.async_copy(rb, spm.at[didx[s]], ssem[b], add=True)
        return carry

    lax.fori_loop(0, nch // 4, quad, 0)
    _scatter(3, 1).wait()
    plsc.subcore_barrier()

    def dump(r, carry):
        o = r0 + r * C
        pltpu.sync_copy(spm.at[pl.ds(o, C), :], rows0)
        pltpu.sync_copy(rows0, out_hbm.at[cid, pl.ds(o, C), :])
        return carry

    lax.fori_loop(0, ROWS_PER_SUB // C, dump, 0)


def _edge_pass(h, ea, src_p, dst_p):
    mesh = plsc.VectorSubcoreMesh(core_axis_name="c", subcore_axis_name="s")
    idx_t = pltpu.VMEM((C,), jnp.int32)
    buf_t = pltpu.VMEM((C, 128), jnp.float32)
    sem_t = pltpu.SemaphoreType.DMA
    return pl.kernel(
        _edge_body,
        out_type=jax.ShapeDtypeStruct((NC, NPAD, 128), jnp.float32),
        mesh=mesh,
        scratch_types=[
            pltpu.VMEM_SHARED((NPAD, 128), jnp.float32),
            idx_t, idx_t, idx_t, idx_t,
            idx_t, idx_t, idx_t, idx_t,
            buf_t, buf_t, buf_t, buf_t,
            sem_t, sem_t, sem_t, sem_t,
            sem_t, sem_t, sem_t, sem_t,
            sem_t, sem_t, sem_t, sem_t,
            sem_t, sem_t,
        ],
    )(h, ea, src_p, dst_p)


_H_SPEC = pl.BlockSpec((NB, 128), lambda i: (i, 0))
_P_SPEC = pl.BlockSpec((NC, NB, 128), lambda i: (0, i, 0))


def _upd1_body(h_ref, p_ref, w_ref, b_ref, eps_ref, o_ref):
    s = (1.0 + eps_ref[0, 0]) * h_ref[...] + p_ref[0] + p_ref[1]
    u = jnp.dot(s, w_ref[...], preferred_element_type=jnp.float32) + b_ref[...]
    o_ref[...] = _elu(u)


def _update1(h, p, wt, b, eps):
    return pl.pallas_call(
        _upd1_body,
        grid=(NGRID,),
        in_specs=[
            _H_SPEC, _P_SPEC,
            pl.BlockSpec((128, 128), lambda i: (0, 0)),
            pl.BlockSpec((1, 128), lambda i: (0, 0)),
            pl.BlockSpec(memory_space=pltpu.SMEM),
        ],
        out_specs=_H_SPEC,
        out_shape=jax.ShapeDtypeStruct((N, 128), jnp.float32),
    )(h, p, wt, b, eps)


def _upd2_body(h_ref, p_ref, w_ref, b_ref, eps_ref, oa_ref, ob_ref):
    s = (1.0 + eps_ref[0, 0]) * h_ref[...] + p_ref[0] + p_ref[1]
    u = jnp.dot(s, w_ref[...], preferred_element_type=jnp.float32) + b_ref[...]
    e = _elu(u)
    oa_ref[...] = e[:, :128]
    ob_ref[...] = e[:, 128:]


def _update2(h, p, wt, b, eps):
    out = jax.ShapeDtypeStruct((N, 128), jnp.float32)
    return pl.pallas_call(
        _upd2_body,
        grid=(NGRID,),
        in_specs=[
            _H_SPEC, _P_SPEC,
            pl.BlockSpec((128, 256), lambda i: (0, 0)),
            pl.BlockSpec((1, 256), lambda i: (0, 0)),
            pl.BlockSpec(memory_space=pltpu.SMEM),
        ],
        out_specs=[_H_SPEC] * 2,
        out_shape=[out] * 2,
    )(h, p, wt, b, eps)


def _upd3_body(ha_ref, hb_ref, pa_ref, pb_ref, w_ref, b_ref, eps_ref,
               batch_ref, o_ref, acc, h3s):
    i = pl.program_id(0)

    @pl.when(i == 0)
    def _():
        acc[...] = jnp.full((G, 128), -jnp.inf, jnp.float32)

    sc = 1.0 + eps_ref[0, 0]
    sa = sc * ha_ref[...] + pa_ref[0] + pa_ref[1]
    sb = sc * hb_ref[...] + pb_ref[0] + pb_ref[1]
    u = (jnp.dot(sa, w_ref[:128, :], preferred_element_type=jnp.float32)
         + jnp.dot(sb, w_ref[128:, :], preferred_element_type=jnp.float32)
         + b_ref[...])
    h3s[...] = _elu(u)

    def group(gi, carry):
        r0 = gi * 8
        g0 = batch_ref[0, 0, r0]
        g7 = batch_ref[0, 0, r0 + 7]

        @pl.when(g0 == g7)
        def _():
            rm = jnp.max(h3s[pl.ds(r0, 8), :], axis=0, keepdims=True)
            acc[pl.ds(g0, 1), :] = jnp.maximum(acc[pl.ds(g0, 1), :], rm)

        @pl.when(g0 != g7)
        def _():
            def row(r, c2):
                g = batch_ref[0, 0, r0 + r]
                rv = h3s[pl.ds(r0 + r, 1), :]
                acc[pl.ds(g, 1), :] = jnp.maximum(acc[pl.ds(g, 1), :], rv)
                return c2

            lax.fori_loop(0, 8, row, 0)

        return carry

    lax.fori_loop(0, NB // 8, group, 0)

    @pl.when(i == pl.num_programs(0) - 1)
    def _():
        o_ref[...] = acc[...]


def _update3_pool(ha, hb, pa, pb, wt, b, eps, batch):
    return pl.pallas_call(
        _upd3_body,
        grid=(NGRID,),
        in_specs=[
            _H_SPEC, _H_SPEC, _P_SPEC, _P_SPEC,
            pl.BlockSpec((256, 128), lambda i: (0, 0)),
            pl.BlockSpec((1, 128), lambda i: (0, 0)),
            pl.BlockSpec(memory_space=pltpu.SMEM),
            pl.BlockSpec((1, 1, NB), lambda i: (i, 0, 0),
                         memory_space=pltpu.SMEM),
        ],
        out_specs=pl.BlockSpec((G, 128), lambda i: (0, 0)),
        out_shape=jax.ShapeDtypeStruct((G, 128), jnp.float32),
        scratch_shapes=[
            pltpu.VMEM((G, 128), jnp.float32),
            pltpu.VMEM((NB, 128), jnp.float32),
        ],
    )(ha, hb, pa, pb, wt, b, eps, batch.reshape(NGRID, 1, NB))


def kernel(x, edge_index, edge_attr, batch,
           W_fc1, b_fc1, W_fc2, b_fc2, W_fc3, b_fc3,
           W1, b1, W2, b2, W3, b3, eps1, eps2, eps3):
    pad = E_PAD - E
    src_p = jnp.concatenate([edge_index[0], jnp.zeros((pad,), jnp.int32)])
    dst_p = jnp.concatenate([edge_index[1], jnp.full((pad,), DUMMY, jnp.int32)])
    ea_pad = jnp.pad(edge_attr, ((0, pad), (0, 0)))

    e1 = jnp.reshape(eps1, (1, 1))
    e2 = jnp.reshape(eps2, (1, 1))
    e3 = jnp.reshape(eps3, (1, 1))

    wcat = jnp.concatenate([W_fc1, W_fc2, W_fc3], axis=0).T
    bcat = jnp.concatenate([b_fc1, b_fc2, b_fc3])[None, :]
    ea1, ea2, ea3a, ea3b = _project(ea_pad, wcat, bcat, 512)

    p1 = _edge_pass(x, ea1, src_p, dst_p)
    h1 = _update1(x, p1, W1.T, b1[None, :], e1)

    p2 = _edge_pass(h1, ea2, src_p, dst_p)
    h2a, h2b = _update2(h1, p2, W2.T, b2[None, :], e2)

    p3a = _edge_pass(h2a, ea3a, src_p, dst_p)
    p3b = _edge_pass(h2b, ea3b, src_p, dst_p)
    out = _update3_pool(h2a, h2b, p3a, p3b, W3.T, b3[None, :], e3, batch)
    return out


# speedup vs baseline: 1.0344x; 1.0344x over previous
import functools

import jax
import jax.numpy as jnp
from jax import lax
from jax.experimental import pallas as pl
from jax.experimental.pallas import tpu as pltpu
from jax.experimental.pallas import tpu_sc as plsc

N = 10000
E = 320000
D = 128
ED = 16
G = 64

NC = 2
NS = 16
NW = NC * NS
C = 64
E_PAD = 327680
EPW0 = 15360
EPW1 = 5120
CH0 = EPW0 // C
CH1 = EPW1 // C
NPAD = 10240
ROWS_PER_SUB = NPAD // NS
DUMMY = N

NB = 1000
NGRID = N // NB


def _elu(u):
    return jnp.where(u > 0, u, jnp.exp(jnp.minimum(u, 0.0)) - 1.0)


EB = 1024


def _proj_body(ea_ref, w_ref, b_ref, *outs):
    u = jnp.dot(ea_ref[...], w_ref[...], preferred_element_type=jnp.float32)
    u = u + b_ref[...]
    for q, o_ref in enumerate(outs):
        o_ref[...] = u[:, q * 128:(q + 1) * 128]


def _project(ea_pad, wt, b, width):
    grid = E_PAD // EB
    nout = width // 128
    out = jax.ShapeDtypeStruct((E_PAD, 128), jnp.float32)
    res = pl.pallas_call(
        _proj_body,
        grid=(grid,),
        in_specs=[
            pl.BlockSpec((EB, ED), lambda i: (i, 0)),
            pl.BlockSpec((ED, width), lambda i: (0, 0)),
            pl.BlockSpec((1, width), lambda i: (0, 0)),
        ],
        out_specs=[pl.BlockSpec((EB, 128), lambda i: (i, 0))] * nout,
        out_shape=[out] * nout,
    )(ea_pad, wt, b)
    return res


def _edge_body(h_hbm, ea_hbm, src_hbm, dst_hbm, out_hbm, spm,
               si0, si1, si2, si3, di0, di1, di2, di3,
               rows0, rows1, eab0, eab1,
               is0, is1, is2, is3, ds0, ds1, ds2, ds3,
               gsem0, gsem1, esem0, esem1, ssem0, ssem1):
    cid = lax.axis_index("c")
    sid = lax.axis_index("s")
    wbase = jnp.where(cid == 0, sid * EPW0, NS * EPW0 + sid * EPW1)
    nch = jnp.where(cid == 0, CH0, CH1)
    sidx = (si0, si1, si2, si3)
    didx = (di0, di1, di2, di3)
    isem = (is0, is1, is2, is3)
    dsem = (ds0, ds1, ds2, ds3)
    rows = (rows0, rows1)
    eab = (eab0, eab1)
    gsem = (gsem0, gsem1)
    esem = (esem0, esem1)
    ssem = (ssem0, ssem1)

    def _idxload(j, s):
        base = wbase + j * C
        a = pltpu.make_async_copy(src_hbm.at[pl.ds(base, C)], sidx[s], isem[s])
        b = pltpu.make_async_copy(dst_hbm.at[pl.ds(base, C)], didx[s], dsem[s])
        return a, b

    def _gather_s(s, b):
        return pltpu.make_async_copy(h_hbm.at[sidx[s]], rows[b], gsem[b])

    def _eacopy(j, b):
        return pltpu.make_async_copy(
            ea_hbm.at[pl.ds(wbase + j * C, C), :], eab[b], esem[b])

    def _scatter(s, b):
        return pltpu.make_async_copy(rows[b], spm.at[didx[s]], ssem[b])

    a0, b0 = _idxload(0, 0)
    a0.start()
    b0.start()
    a1, b1 = _idxload(1, 1)
    a1.start()
    b1.start()

    zeros16 = jnp.zeros((16,), jnp.float32)

    def zrow(i, carry):
        for k in range(8):
            rows0[i, pl.ds(k * 16, 16)] = zeros16
        return carry

    lax.fori_loop(0, C, zrow, 0)

    r0 = sid * ROWS_PER_SUB

    def zspm(r, carry):
        pltpu.sync_copy(rows0, spm.at[pl.ds(r0 + r * C, C), :])
        return carry

    lax.fori_loop(0, ROWS_PER_SUB // C, zspm, 0)
    plsc.subcore_barrier()

    a0.wait()
    _gather_s(0, 0).start()
    _eacopy(0, 0).start()

    def quad(jq, carry):
        for s in range(4):
            j = jq * 4 + s
            b = s % 2
            nb = 1 - b
            ns = (s + 1) % 4

            @pl.when(j >= 1)
            def _():
                _scatter((s + 3) % 4, nb).wait()

            @pl.when(j < nch - 2)
            def _():
                a, bb = _idxload(j + 2, (s + 2) % 4)
                a.start()
                bb.start()

            @pl.when(j < nch - 1)
            def _():
                pltpu.make_async_copy(
                    src_hbm.at[pl.ds(wbase + (j + 1) * C, C)],
                    sidx[ns], isem[ns]).wait()
                _gather_s(ns, nb).start()
                _eacopy(j + 1, nb).start()

            _gather_s(s, b).wait()
            _eacopy(j, b).wait()

            rb = rows[b]
            eb = eab[b]

            def comp(i, inner):
                for k in range(8):
                    sl = pl.ds(k * 16, 16)
                    rb[i, sl] = jnp.maximum(rb[i, sl] + eb[i, sl], 0.0)
                return inner

            lax.fori_loop(0, C, comp, 0)

            pltpu.make_async_copy(
                dst_hbm.at[pl.ds(wbase + j * C, C)], didx[s], dsem[s]).wait()
            pltpu.async_copy(rb, spm.at[didx[s]], ssem[b], add=True)
        return carry

    lax.fori_loop(0, nch // 4, quad, 0)
    _scatter(3, 1).wait()
    plsc.subcore_barrier()

    def dump(r, carry):
        o = r0 + r * C
        pltpu.sync_copy(spm.at[pl.ds(o, C), :], rows0)
        pltpu.sync_copy(rows0, out_hbm.at[cid, pl.ds(o, C), :])
        return carry

    lax.fori_loop(0, ROWS_PER_SUB // C, dump, 0)


def _edge_pass(h, ea, src_p, dst_p):
    mesh = plsc.VectorSubcoreMesh(core_axis_name="c", subcore_axis_name="s")
    idx_t = pltpu.VMEM((C,), jnp.int32)
    buf_t = pltpu.VMEM((C, 128), jnp.float32)
    sem_t = pltpu.SemaphoreType.DMA
    return pl.kernel(
        _edge_body,
        out_type=jax.ShapeDtypeStruct((NC, NPAD, 128), jnp.float32),
        mesh=mesh,
        scratch_types=[
            pltpu.VMEM_SHARED((NPAD, 128), jnp.float32),
            idx_t, idx_t, idx_t, idx_t,
            idx_t, idx_t, idx_t, idx_t,
            buf_t, buf_t, buf_t, buf_t,
            sem_t, sem_t, sem_t, sem_t,
            sem_t, sem_t, sem_t, sem_t,
            sem_t, sem_t, sem_t, sem_t,
            sem_t, sem_t,
        ],
    )(h, ea, src_p, dst_p)


_H_SPEC = pl.BlockSpec((NB, 128), lambda i: (i, 0))
_P_SPEC = pl.BlockSpec((NC, NB, 128), lambda i: (0, i, 0))


def _upd1_body(h_ref, p_ref, w_ref, b_ref, eps_ref, o_ref):
    s = (1.0 + eps_ref[0, 0]) * h_ref[...] + p_ref[0] + p_ref[1]
    u = jnp.dot(s, w_ref[...], preferred_element_type=jnp.float32) + b_ref[...]
    o_ref[...] = _elu(u)


def _update1(h, p, wt, b, eps):
    return pl.pallas_call(
        _upd1_body,
        grid=(NGRID,),
        in_specs=[
            _H_SPEC, _P_SPEC,
            pl.BlockSpec((128, 128), lambda i: (0, 0)),
            pl.BlockSpec((1, 128), lambda i: (0, 0)),
            pl.BlockSpec(memory_space=pltpu.SMEM),
        ],
        out_specs=_H_SPEC,
        out_shape=jax.ShapeDtypeStruct((N, 128), jnp.float32),
    )(h, p, wt, b, eps)


def _upd2_body(h_ref, p_ref, w_ref, b_ref, eps_ref, oa_ref, ob_ref):
    s = (1.0 + eps_ref[0, 0]) * h_ref[...] + p_ref[0] + p_ref[1]
    u = jnp.dot(s, w_ref[...], preferred_element_type=jnp.float32) + b_ref[...]
    e = _elu(u)
    oa_ref[...] = e[:, :128]
    ob_ref[...] = e[:, 128:]


def _update2(h, p, wt, b, eps):
    out = jax.ShapeDtypeStruct((N, 128), jnp.float32)
    return pl.pallas_call(
        _upd2_body,
        grid=(NGRID,),
        in_specs=[
            _H_SPEC, _P_SPEC,
            pl.BlockSpec((128, 256), lambda i: (0, 0)),
            pl.BlockSpec((1, 256), lambda i: (0, 0)),
            pl.BlockSpec(memory_space=pltpu.SMEM),
        ],
        out_specs=[_H_SPEC] * 2,
        out_shape=[out] * 2,
    )(h, p, wt, b, eps)


def _upd3_body(ha_ref, hb_ref, pa_ref, pb_ref, w_ref, b_ref, eps_ref,
               batch_ref, o_ref, acc, h3s):
    i = pl.program_id(0)

    @pl.when(i == 0)
    def _():
        acc[...] = jnp.full((G, 128), -jnp.inf, jnp.float32)

    sc = 1.0 + eps_ref[0, 0]
    sa = sc * ha_ref[...] + pa_ref[0] + pa_ref[1]
    sb = sc * hb_ref[...] + pb_ref[0] + pb_ref[1]
    u = (jnp.dot(sa, w_ref[:128, :], preferred_element_type=jnp.float32)
         + jnp.dot(sb, w_ref[128:, :], preferred_element_type=jnp.float32)
         + b_ref[...])
    h3s[...] = _elu(u)

    def group(gi, carry):
        r0 = gi * 8
        g0 = batch_ref[0, 0, r0]
        g7 = batch_ref[0, 0, r0 + 7]

        @pl.when(g0 == g7)
        def _():
            rm = jnp.max(h3s[pl.ds(r0, 8), :], axis=0, keepdims=True)
            acc[pl.ds(g0, 1), :] = jnp.maximum(acc[pl.ds(g0, 1), :], rm)

        @pl.when(g0 != g7)
        def _():
            def row(r, c2):
                g = batch_ref[0, 0, r0 + r]
                rv = h3s[pl.ds(r0 + r, 1), :]
                acc[pl.ds(g, 1), :] = jnp.maximum(acc[pl.ds(g, 1), :], rv)
                return c2

            lax.fori_loop(0, 8, row, 0)

        return carry

    lax.fori_loop(0, NB // 8, group, 0)

    @pl.when(i == pl.num_programs(0) - 1)
    def _():
        o_ref[...] = acc[...]


def _update3_pool(ha, hb, pa, pb, wt, b, eps, batch):
    return pl.pallas_call(
        _upd3_body,
        grid=(NGRID,),
        in_specs=[
            _H_SPEC, _H_SPEC, _P_SPEC, _P_SPEC,
            pl.BlockSpec((256, 128), lambda i: (0, 0)),
            pl.BlockSpec((1, 128), lambda i: (0, 0)),
            pl.BlockSpec(memory_space=pltpu.SMEM),
            pl.BlockSpec((1, 1, NB), lambda i: (i, 0, 0),
                         memory_space=pltpu.SMEM),
        ],
        out_specs=pl.BlockSpec((G, 128), lambda i: (0, 0)),
        out_shape=jax.ShapeDtypeStruct((G, 128), jnp.float32),
        scratch_shapes=[
            pltpu.VMEM((G, 128), jnp.float32),
            pltpu.VMEM((NB, 128), jnp.float32),
        ],
    )(ha, hb, pa, pb, wt, b, eps, batch.reshape(NGRID, 1, NB))


def kernel(x, edge_index, edge_attr, batch,
           W_fc1, b_fc1, W_fc2, b_fc2, W_fc3, b_fc3,
           W1, b1, W2, b2, W3, b3, eps1, eps2, eps3):
    pad = E_PAD - E
    src_p = jnp.concatenate([edge_index[0], jnp.zeros((pad,), jnp.int32)])
    dst_p = jnp.concatenate([edge_index[1], jnp.full((pad,), DUMMY, jnp.int32)])
    ea_pad = jnp.pad(edge_attr, ((0, pad), (0, 0)))

    e1 = jnp.reshape(eps1, (1, 1))
    e2 = jnp.reshape(eps2, (1, 1))
    e3 = jnp.reshape(eps3, (1, 1))

    wcat = jnp.concatenate([W_fc1, W_fc2, W_fc3], axis=0).T
    bcat = jnp.concatenate([b_fc1, b_fc2, b_fc3])[None, :]
    ea1, ea2, ea3a, ea3b = _project(ea_pad, wcat, bcat, 512)

    p1 = _edge_pass(x, ea1, src_p, dst_p)
    h1 = _update1(x, p1, W1.T, b1[None, :], e1)

    p2 = _edge_pass(h1, ea2, src_p, dst_p)
    h2a, h2b = _update2(h1, p2, W2.T, b2[None, :], e2)

    p3a = _edge_pass(h2a, ea3a, src_p, dst_p)
    p3b = _edge_pass(h2b, ea3b, src_p, dst_p)
    out = _update3_pool(h2a, h2b, p3a, p3b, W3.T, b3[None, :], e3, batch)
    return out


# speedup vs baseline: 1.0425x; 1.0078x over previous
import functools

import jax
import jax.numpy as jnp
from jax import lax
from jax.experimental import pallas as pl
from jax.experimental.pallas import tpu as pltpu
from jax.experimental.pallas import tpu_sc as plsc

N = 10000
E = 320000
D = 128
ED = 16
G = 64

NC = 2
NS = 16
NW = NC * NS
C = 64
E_PAD = 327680
EPW0 = 16384
EPW1 = 4096
CH0 = EPW0 // C
CH1 = EPW1 // C
NPAD = 10240
ROWS_PER_SUB = NPAD // NS
DUMMY = N

NB = 1000
NGRID = N // NB


def _elu(u):
    return jnp.where(u > 0, u, jnp.exp(jnp.minimum(u, 0.0)) - 1.0)


EB = 1024


def _proj_body(ea_ref, w_ref, b_ref, *outs):
    u = jnp.dot(ea_ref[...], w_ref[...], preferred_element_type=jnp.float32)
    u = u + b_ref[...]
    for q, o_ref in enumerate(outs):
        o_ref[...] = u[:, q * 128:(q + 1) * 128]


def _project(ea_pad, wt, b, width):
    grid = E_PAD // EB
    nout = width // 128
    out = jax.ShapeDtypeStruct((E_PAD, 128), jnp.float32)
    res = pl.pallas_call(
        _proj_body,
        grid=(grid,),
        in_specs=[
            pl.BlockSpec((EB, ED), lambda i: (i, 0)),
            pl.BlockSpec((ED, width), lambda i: (0, 0)),
            pl.BlockSpec((1, width), lambda i: (0, 0)),
        ],
        out_specs=[pl.BlockSpec((EB, 128), lambda i: (i, 0))] * nout,
        out_shape=[out] * nout,
    )(ea_pad, wt, b)
    return res


def _edge_body(h_hbm, ea_hbm, src_hbm, dst_hbm, out_hbm, spm,
               si0, si1, si2, si3, di0, di1, di2, di3,
               rows0, rows1, eab0, eab1,
               is0, is1, is2, is3, ds0, ds1, ds2, ds3,
               gsem0, gsem1, esem0, esem1, ssem0, ssem1):
    cid = lax.axis_index("c")
    sid = lax.axis_index("s")
    wbase = jnp.where(cid == 0, sid * EPW0, NS * EPW0 + sid * EPW1)
    nch = jnp.where(cid == 0, CH0, CH1)
    sidx = (si0, si1, si2, si3)
    didx = (di0, di1, di2, di3)
    isem = (is0, is1, is2, is3)
    dsem = (ds0, ds1, ds2, ds3)
    rows = (rows0, rows1)
    eab = (eab0, eab1)
    gsem = (gsem0, gsem1)
    esem = (esem0, esem1)
    ssem = (ssem0, ssem1)

    def _idxload(j, s):
        base = wbase + j * C
        a = pltpu.make_async_copy(src_hbm.at[pl.ds(base, C)], sidx[s], isem[s])
        b = pltpu.make_async_copy(dst_hbm.at[pl.ds(base, C)], didx[s], dsem[s])
        return a, b

    def _gather_s(s, b):
        return pltpu.make_async_copy(h_hbm.at[sidx[s]], rows[b], gsem[b])

    def _eacopy(j, b):
        return pltpu.make_async_copy(
            ea_hbm.at[pl.ds(wbase + j * C, C), :], eab[b], esem[b])

    def _scatter(s, b):
        return pltpu.make_async_copy(rows[b], spm.at[didx[s]], ssem[b])

    a0, b0 = _idxload(0, 0)
    a0.start()
    b0.start()
    a1, b1 = _idxload(1, 1)
    a1.start()
    b1.start()

    zeros16 = jnp.zeros((16,), jnp.float32)

    def zrow(i, carry):
        for k in range(8):
            rows0[i, pl.ds(k * 16, 16)] = zeros16
        return carry

    lax.fori_loop(0, C, zrow, 0)

    r0 = sid * ROWS_PER_SUB

    def zspm(r, carry):
        pltpu.sync_copy(rows0, spm.at[pl.ds(r0 + r * C, C), :])
        return carry

    lax.fori_loop(0, ROWS_PER_SUB // C, zspm, 0)
    plsc.subcore_barrier()

    a0.wait()
    _gather_s(0, 0).start()
    _eacopy(0, 0).start()

    def quad(jq, carry):
        for s in range(4):
            j = jq * 4 + s
            b = s % 2
            nb = 1 - b
            ns = (s + 1) % 4

            @pl.when(j >= 1)
            def _():
                _scatter((s + 3) % 4, nb).wait()

            @pl.when(j < nch - 2)
            def _():
                a, bb = _idxload(j + 2, (s + 2) % 4)
                a.start()
                bb.start()

            @pl.when(j < nch - 1)
            def _():
                pltpu.make_async_copy(
                    src_hbm.at[pl.ds(wbase + (j + 1) * C, C)],
                    sidx[ns], isem[ns]).wait()
                _gather_s(ns, nb).start()
                _eacopy(j + 1, nb).start()

            _gather_s(s, b).wait()
            _eacopy(j, b).wait()

            rb = rows[b]
            eb = eab[b]

            def comp(i, inner):
                for k in range(8):
                    sl = pl.ds(k * 16, 16)
                    rb[i, sl] = jnp.maximum(rb[i, sl] + eb[i, sl], 0.0)
                return inner

            lax.fori_loop(0, C, comp, 0)

            pltpu.make_async_copy(
                dst_hbm.at[pl.ds(wbase + j * C, C)], didx[s], dsem[s]).wait()
            pltpu.async_copy(rb, spm.at[didx[s]], ssem[b], add=True)
        return carry

    lax.fori_loop(0, nch // 4, quad, 0)
    _scatter(3, 1).wait()
    plsc.subcore_barrier()

    def dump(r, carry):
        o = r0 + r * C
        pltpu.sync_copy(spm.at[pl.ds(o, C), :], rows0)
        pltpu.sync_copy(rows0, out_hbm.at[cid, pl.ds(o, C), :])
        return carry

    lax.fori_loop(0, ROWS_PER_SUB // C, dump, 0)


def _edge_pass(h, ea, src_p, dst_p):
    mesh = plsc.VectorSubcoreMesh(core_axis_name="c", subcore_axis_name="s")
    idx_t = pltpu.VMEM((C,), jnp.int32)
    buf_t = pltpu.VMEM((C, 128), jnp.float32)
    sem_t = pltpu.SemaphoreType.DMA
    return pl.kernel(
        _edge_body,
        out_type=jax.ShapeDtypeStruct((NC, NPAD, 128), jnp.float32),
        mesh=mesh,
        scratch_types=[
            pltpu.VMEM_SHARED((NPAD, 128), jnp.float32),
            idx_t, idx_t, idx_t, idx_t,
            idx_t, idx_t, idx_t, idx_t,
            buf_t, buf_t, buf_t, buf_t,
            sem_t, sem_t, sem_t, sem_t,
            sem_t, sem_t, sem_t, sem_t,
            sem_t, sem_t, sem_t, sem_t,
            sem_t, sem_t,
        ],
    )(h, ea, src_p, dst_p)


_H_SPEC = pl.BlockSpec((NB, 128), lambda i: (i, 0))
_P_SPEC = pl.BlockSpec((NC, NB, 128), lambda i: (0, i, 0))


def _upd1_body(h_ref, p_ref, w_ref, b_ref, eps_ref, o_ref):
    s = (1.0 + eps_ref[0, 0]) * h_ref[...] + p_ref[0] + p_ref[1]
    u = jnp.dot(s, w_ref[...], preferred_element_type=jnp.float32) + b_ref[...]
    o_ref[...] = _elu(u)


def _update1(h, p, wt, b, eps):
    return pl.pallas_call(
        _upd1_body,
        grid=(NGRID,),
        in_specs=[
            _H_SPEC, _P_SPEC,
            pl.BlockSpec((128, 128), lambda i: (0, 0)),
            pl.BlockSpec((1, 128), lambda i: (0, 0)),
            pl.BlockSpec(memory_space=pltpu.SMEM),
        ],
        out_specs=_H_SPEC,
        out_shape=jax.ShapeDtypeStruct((N, 128), jnp.float32),
    )(h, p, wt, b, eps)


def _upd2_body(h_ref, p_ref, w_ref, b_ref, eps_ref, oa_ref, ob_ref):
    s = (1.0 + eps_ref[0, 0]) * h_ref[...] + p_ref[0] + p_ref[1]
    u = jnp.dot(s, w_ref[...], preferred_element_type=jnp.float32) + b_ref[...]
    e = _elu(u)
    oa_ref[...] = e[:, :128]
    ob_ref[...] = e[:, 128:]


def _update2(h, p, wt, b, eps):
    out = jax.ShapeDtypeStruct((N, 128), jnp.float32)
    return pl.pallas_call(
        _upd2_body,
        grid=(NGRID,),
        in_specs=[
            _H_SPEC, _P_SPEC,
            pl.BlockSpec((128, 256), lambda i: (0, 0)),
            pl.BlockSpec((1, 256), lambda i: (0, 0)),
            pl.BlockSpec(memory_space=pltpu.SMEM),
        ],
        out_specs=[_H_SPEC] * 2,
        out_shape=[out] * 2,
    )(h, p, wt, b, eps)


def _upd3_body(ha_ref, hb_ref, pa_ref, pb_ref, w_ref, b_ref, eps_ref,
               batch_ref, o_ref, acc, h3s):
    i = pl.program_id(0)

    @pl.when(i == 0)
    def _():
        acc[...] = jnp.full((G, 128), -jnp.inf, jnp.float32)

    sc = 1.0 + eps_ref[0, 0]
    sa = sc * ha_ref[...] + pa_ref[0] + pa_ref[1]
    sb = sc * hb_ref[...] + pb_ref[0] + pb_ref[1]
    u = (jnp.dot(sa, w_ref[:128, :], preferred_element_type=jnp.float32)
         + jnp.dot(sb, w_ref[128:, :], preferred_element_type=jnp.float32)
         + b_ref[...])
    h3s[...] = _elu(u)

    def group(gi, carry):
        r0 = gi * 8
        g0 = batch_ref[0, 0, r0]
        g7 = batch_ref[0, 0, r0 + 7]

        @pl.when(g0 == g7)
        def _():
            rm = jnp.max(h3s[pl.ds(r0, 8), :], axis=0, keepdims=True)
            acc[pl.ds(g0, 1), :] = jnp.maximum(acc[pl.ds(g0, 1), :], rm)

        @pl.when(g0 != g7)
        def _():
            def row(r, c2):
                g = batch_ref[0, 0, r0 + r]
                rv = h3s[pl.ds(r0 + r, 1), :]
                acc[pl.ds(g, 1), :] = jnp.maximum(acc[pl.ds(g, 1), :], rv)
                return c2

            lax.fori_loop(0, 8, row, 0)

        return carry

    lax.fori_loop(0, NB // 8, group, 0)

    @pl.when(i == pl.num_programs(0) - 1)
    def _():
        o_ref[...] = acc[...]


def _update3_pool(ha, hb, pa, pb, wt, b, eps, batch):
    return pl.pallas_call(
        _upd3_body,
        grid=(NGRID,),
        in_specs=[
            _H_SPEC, _H_SPEC, _P_SPEC, _P_SPEC,
            pl.BlockSpec((256, 128), lambda i: (0, 0)),
            pl.BlockSpec((1, 128), lambda i: (0, 0)),
            pl.BlockSpec(memory_space=pltpu.SMEM),
            pl.BlockSpec((1, 1, NB), lambda i: (i, 0, 0),
                         memory_space=pltpu.SMEM),
        ],
        out_specs=pl.BlockSpec((G, 128), lambda i: (0, 0)),
        out_shape=jax.ShapeDtypeStruct((G, 128), jnp.float32),
        scratch_shapes=[
            pltpu.VMEM((G, 128), jnp.float32),
            pltpu.VMEM((NB, 128), jnp.float32),
        ],
    )(ha, hb, pa, pb, wt, b, eps, batch.reshape(NGRID, 1, NB))


def kernel(x, edge_index, edge_attr, batch,
           W_fc1, b_fc1, W_fc2, b_fc2, W_fc3, b_fc3,
           W1, b1, W2, b2, W3, b3, eps1, eps2, eps3):
    pad = E_PAD - E
    src_p = jnp.concatenate([edge_index[0], jnp.zeros((pad,), jnp.int32)])
    dst_p = jnp.concatenate([edge_index[1], jnp.full((pad,), DUMMY, jnp.int32)])
    ea_pad = jnp.pad(edge_attr, ((0, pad), (0, 0)))

    e1 = jnp.reshape(eps1, (1, 1))
    e2 = jnp.reshape(eps2, (1, 1))
    e3 = jnp.reshape(eps3, (1, 1))

    wcat = jnp.concatenate([W_fc1, W_fc2, W_fc3], axis=0).T
    bcat = jnp.concatenate([b_fc1, b_fc2, b_fc3])[None, :]
    ea1, ea2, ea3a, ea3b = _project(ea_pad, wcat, bcat, 512)

    p1 = _edge_pass(x, ea1, src_p, dst_p)
    h1 = _update1(x, p1, W1.T, b1[None, :], e1)

    p2 = _edge_pass(h1, ea2, src_p, dst_p)
    h2a, h2b = _update2(h1, p2, W2.T, b2[None, :], e2)

    p3a = _edge_pass(h2a, ea3a, src_p, dst_p)
    p3b = _edge_pass(h2b, ea3b, src_p, dst_p)
    out = _update3_pool(h2a, h2b, p3a, p3b, W3.T, b3[None, :], e3, batch)
    return out


# speedup vs baseline: 1.0722x; 1.0285x over previous
import functools

import jax
import jax.numpy as jnp
from jax import lax
from jax.experimental import pallas as pl
from jax.experimental.pallas import tpu as pltpu
from jax.experimental.pallas import tpu_sc as plsc

N = 10000
E = 320000
D = 128
ED = 16
G = 64

NC = 2
NS = 16
NW = NC * NS
C = 64
E_PAD = 327680
EPW0 = 18432
EPW1 = 2048
CH0 = EPW0 // C
CH1 = EPW1 // C
NPAD = 10240
ROWS_PER_SUB = NPAD // NS
DUMMY = N

NB = 1000
NGRID = N // NB


def _elu(u):
    return jnp.where(u > 0, u, jnp.exp(jnp.minimum(u, 0.0)) - 1.0)


EB = 1024


def _proj_body(ea_ref, w_ref, b_ref, *outs):
    u = jnp.dot(ea_ref[...], w_ref[...], preferred_element_type=jnp.float32)
    u = u + b_ref[...]
    for q, o_ref in enumerate(outs):
        o_ref[...] = u[:, q * 128:(q + 1) * 128]


def _project(ea_pad, wt, b, width):
    grid = E_PAD // EB
    nout = width // 128
    out = jax.ShapeDtypeStruct((E_PAD, 128), jnp.float32)
    res = pl.pallas_call(
        _proj_body,
        grid=(grid,),
        in_specs=[
            pl.BlockSpec((EB, ED), lambda i: (i, 0)),
            pl.BlockSpec((ED, width), lambda i: (0, 0)),
            pl.BlockSpec((1, width), lambda i: (0, 0)),
        ],
        out_specs=[pl.BlockSpec((EB, 128), lambda i: (i, 0))] * nout,
        out_shape=[out] * nout,
    )(ea_pad, wt, b)
    return res


def _edge_body(h_hbm, ea_hbm, src_hbm, dst_hbm, out_hbm, spm,
               si0, si1, si2, si3, di0, di1, di2, di3,
               rows0, rows1, eab0, eab1,
               is0, is1, is2, is3, ds0, ds1, ds2, ds3,
               gsem0, gsem1, esem0, esem1, ssem0, ssem1):
    cid = lax.axis_index("c")
    sid = lax.axis_index("s")
    wbase = jnp.where(cid == 0, sid * EPW0, NS * EPW0 + sid * EPW1)
    nch = jnp.where(cid == 0, CH0, CH1)
    sidx = (si0, si1, si2, si3)
    didx = (di0, di1, di2, di3)
    isem = (is0, is1, is2, is3)
    dsem = (ds0, ds1, ds2, ds3)
    rows = (rows0, rows1)
    eab = (eab0, eab1)
    gsem = (gsem0, gsem1)
    esem = (esem0, esem1)
    ssem = (ssem0, ssem1)

    def _idxload(j, s):
        base = wbase + j * C
        a = pltpu.make_async_copy(src_hbm.at[pl.ds(base, C)], sidx[s], isem[s])
        b = pltpu.make_async_copy(dst_hbm.at[pl.ds(base, C)], didx[s], dsem[s])
        return a, b

    def _gather_s(s, b):
        return pltpu.make_async_copy(h_hbm.at[sidx[s]], rows[b], gsem[b])

    def _eacopy(j, b):
        return pltpu.make_async_copy(
            ea_hbm.at[pl.ds(wbase + j * C, C), :], eab[b], esem[b])

    def _scatter(s, b):
        return pltpu.make_async_copy(rows[b], spm.at[didx[s]], ssem[b])

    a0, b0 = _idxload(0, 0)
    a0.start()
    b0.start()
    a1, b1 = _idxload(1, 1)
    a1.start()
    b1.start()

    zeros16 = jnp.zeros((16,), jnp.float32)

    def zrow(i, carry):
        for k in range(8):
            rows0[i, pl.ds(k * 16, 16)] = zeros16
        return carry

    lax.fori_loop(0, C, zrow, 0)

    r0 = sid * ROWS_PER_SUB

    def zspm(r, carry):
        pltpu.sync_copy(rows0, spm.at[pl.ds(r0 + r * C, C), :])
        return carry

    lax.fori_loop(0, ROWS_PER_SUB // C, zspm, 0)
    plsc.subcore_barrier()

    a0.wait()
    _gather_s(0, 0).start()
    _eacopy(0, 0).start()

    def quad(jq, carry):
        for s in range(4):
            j = jq * 4 + s
            b = s % 2
            nb = 1 - b
            ns = (s + 1) % 4

            @pl.when(j >= 1)
            def _():
                _scatter((s + 3) % 4, nb).wait()

            @pl.when(j < nch - 2)
            def _():
                a, bb = _idxload(j + 2, (s + 2) % 4)
                a.start()
                bb.start()

            @pl.when(j < nch - 1)
            def _():
                pltpu.make_async_copy(
                    src_hbm.at[pl.ds(wbase + (j + 1) * C, C)],
                    sidx[ns], isem[ns]).wait()
                _gather_s(ns, nb).start()
                _eacopy(j + 1, nb).start()

            _gather_s(s, b).wait()
            _eacopy(j, b).wait()

            rb = rows[b]
            eb = eab[b]

            def comp(i, inner):
                for k in range(8):
                    sl = pl.ds(k * 16, 16)
                    rb[i, sl] = jnp.maximum(rb[i, sl] + eb[i, sl], 0.0)
                return inner

            lax.fori_loop(0, C, comp, 0)

            pltpu.make_async_copy(
                dst_hbm.at[pl.ds(wbase + j * C, C)], didx[s], dsem[s]).wait()
            pltpu.async_copy(rb, spm.at[didx[s]], ssem[b], add=True)
        return carry

    lax.fori_loop(0, nch // 4, quad, 0)
    _scatter(3, 1).wait()
    plsc.subcore_barrier()

    def dump(r, carry):
        o = r0 + r * C
        pltpu.sync_copy(spm.at[pl.ds(o, C), :], rows0)
        pltpu.sync_copy(rows0, out_hbm.at[cid, pl.ds(o, C), :])
        return carry

    lax.fori_loop(0, ROWS_PER_SUB // C, dump, 0)


def _edge_pass(h, ea, src_p, dst_p):
    mesh = plsc.VectorSubcoreMesh(core_axis_name="c", subcore_axis_name="s")
    idx_t = pltpu.VMEM((C,), jnp.int32)
    buf_t = pltpu.VMEM((C, 128), jnp.float32)
    sem_t = pltpu.SemaphoreType.DMA
    return pl.kernel(
        _edge_body,
        out_type=jax.ShapeDtypeStruct((NC, NPAD, 128), jnp.float32),
        mesh=mesh,
        scratch_types=[
            pltpu.VMEM_SHARED((NPAD, 128), jnp.float32),
            idx_t, idx_t, idx_t, idx_t,
            idx_t, idx_t, idx_t, idx_t,
            buf_t, buf_t, buf_t, buf_t,
            sem_t, sem_t, sem_t, sem_t,
            sem_t, sem_t, sem_t, sem_t,
            sem_t, sem_t, sem_t, sem_t,
            sem_t, sem_t,
        ],
    )(h, ea, src_p, dst_p)


_H_SPEC = pl.BlockSpec((NB, 128), lambda i: (i, 0))
_P_SPEC = pl.BlockSpec((NC, NB, 128), lambda i: (0, i, 0))


def _upd1_body(h_ref, p_ref, w_ref, b_ref, eps_ref, o_ref):
    s = (1.0 + eps_ref[0, 0]) * h_ref[...] + p_ref[0] + p_ref[1]
    u = jnp.dot(s, w_ref[...], preferred_element_type=jnp.float32) + b_ref[...]
    o_ref[...] = _elu(u)


def _update1(h, p, wt, b, eps):
    return pl.pallas_call(
        _upd1_body,
        grid=(NGRID,),
        in_specs=[
            _H_SPEC, _P_SPEC,
            pl.BlockSpec((128, 128), lambda i: (0, 0)),
            pl.BlockSpec((1, 128), lambda i: (0, 0)),
            pl.BlockSpec(memory_space=pltpu.SMEM),
        ],
        out_specs=_H_SPEC,
        out_shape=jax.ShapeDtypeStruct((N, 128), jnp.float32),
    )(h, p, wt, b, eps)


def _upd2_body(h_ref, p_ref, w_ref, b_ref, eps_ref, oa_ref, ob_ref):
    s = (1.0 + eps_ref[0, 0]) * h_ref[...] + p_ref[0] + p_ref[1]
    u = jnp.dot(s, w_ref[...], preferred_element_type=jnp.float32) + b_ref[...]
    e = _elu(u)
    oa_ref[...] = e[:, :128]
    ob_ref[...] = e[:, 128:]


def _update2(h, p, wt, b, eps):
    out = jax.ShapeDtypeStruct((N, 128), jnp.float32)
    return pl.pallas_call(
        _upd2_body,
        grid=(NGRID,),
        in_specs=[
            _H_SPEC, _P_SPEC,
            pl.BlockSpec((128, 256), lambda i: (0, 0)),
            pl.BlockSpec((1, 256), lambda i: (0, 0)),
            pl.BlockSpec(memory_space=pltpu.SMEM),
        ],
        out_specs=[_H_SPEC] * 2,
        out_shape=[out] * 2,
    )(h, p, wt, b, eps)


def _upd3_body(ha_ref, hb_ref, pa_ref, pb_ref, w_ref, b_ref, eps_ref,
               batch_ref, o_ref, acc, h3s):
    i = pl.program_id(0)

    @pl.when(i == 0)
    def _():
        acc[...] = jnp.full((G, 128), -jnp.inf, jnp.float32)

    sc = 1.0 + eps_ref[0, 0]
    sa = sc * ha_ref[...] + pa_ref[0] + pa_ref[1]
    sb = sc * hb_ref[...] + pb_ref[0] + pb_ref[1]
    u = (jnp.dot(sa, w_ref[:128, :], preferred_element_type=jnp.float32)
         + jnp.dot(sb, w_ref[128:, :], preferred_element_type=jnp.float32)
         + b_ref[...])
    h3s[...] = _elu(u)

    def group(gi, carry):
        r0 = gi * 8
        g0 = batch_ref[0, 0, r0]
        g7 = batch_ref[0, 0, r0 + 7]

        @pl.when(g0 == g7)
        def _():
            rm = jnp.max(h3s[pl.ds(r0, 8), :], axis=0, keepdims=True)
            acc[pl.ds(g0, 1), :] = jnp.maximum(acc[pl.ds(g0, 1), :], rm)

        @pl.when(g0 != g7)
        def _():
            def row(r, c2):
                g = batch_ref[0, 0, r0 + r]
                rv = h3s[pl.ds(r0 + r, 1), :]
                acc[pl.ds(g, 1), :] = jnp.maximum(acc[pl.ds(g, 1), :], rv)
                return c2

            lax.fori_loop(0, 8, row, 0)

        return carry

    lax.fori_loop(0, NB // 8, group, 0)

    @pl.when(i == pl.num_programs(0) - 1)
    def _():
        o_ref[...] = acc[...]


def _update3_pool(ha, hb, pa, pb, wt, b, eps, batch):
    return pl.pallas_call(
        _upd3_body,
        grid=(NGRID,),
        in_specs=[
            _H_SPEC, _H_SPEC, _P_SPEC, _P_SPEC,
            pl.BlockSpec((256, 128), lambda i: (0, 0)),
            pl.BlockSpec((1, 128), lambda i: (0, 0)),
            pl.BlockSpec(memory_space=pltpu.SMEM),
            pl.BlockSpec((1, 1, NB), lambda i: (i, 0, 0),
                         memory_space=pltpu.SMEM),
        ],
        out_specs=pl.BlockSpec((G, 128), lambda i: (0, 0)),
        out_shape=jax.ShapeDtypeStruct((G, 128), jnp.float32),
        scratch_shapes=[
            pltpu.VMEM((G, 128), jnp.float32),
            pltpu.VMEM((NB, 128), jnp.float32),
        ],
    )(ha, hb, pa, pb, wt, b, eps, batch.reshape(NGRID, 1, NB))


def kernel(x, edge_index, edge_attr, batch,
           W_fc1, b_fc1, W_fc2, b_fc2, W_fc3, b_fc3,
           W1, b1, W2, b2, W3, b3, eps1, eps2, eps3):
    pad = E_PAD - E
    src_p = jnp.concatenate([edge_index[0], jnp.zeros((pad,), jnp.int32)])
    dst_p = jnp.concatenate([edge_index[1], jnp.full((pad,), DUMMY, jnp.int32)])
    ea_pad = jnp.pad(edge_attr, ((0, pad), (0, 0)))

    e1 = jnp.reshape(eps1, (1, 1))
    e2 = jnp.reshape(eps2, (1, 1))
    e3 = jnp.reshape(eps3, (1, 1))

    wcat = jnp.concatenate([W_fc1, W_fc2, W_fc3], axis=0).T
    bcat = jnp.concatenate([b_fc1, b_fc2, b_fc3])[None, :]
    ea1, ea2, ea3a, ea3b = _project(ea_pad, wcat, bcat, 512)

    p1 = _edge_pass(x, ea1, src_p, dst_p)
    h1 = _update1(x, p1, W1.T, b1[None, :], e1)

    p2 = _edge_pass(h1, ea2, src_p, dst_p)
    h2a, h2b = _update2(h1, p2, W2.T, b2[None, :], e2)

    p3a = _edge_pass(h2a, ea3a, src_p, dst_p)
    p3b = _edge_pass(h2b, ea3b, src_p, dst_p)
    out = _update3_pool(h2a, h2b, p3a, p3b, W3.T, b3[None, :], e3, batch)
    return out
